# Initial kernel scaffold; baseline (speedup 1.0000x reference)
import jax
import jax.numpy as jnp
from jax.experimental import pallas as pl


def kernel(X, S, edge_index, emb, W_rad, W_e1, W_e2, W_x, W_h1, W_h2, W_m1, W_m2, W_r1, W_r2):
    raise NotImplementedError("write your pallas kernel here")


# speedup vs baseline: 23.5998x; 23.5998x over previous
import functools

import numpy as np
import jax
import jax.numpy as jnp
from jax import lax
from jax.experimental import pallas as pl
from jax.experimental.pallas import tpu as pltpu
from jax.experimental.pallas import tpu_sc as plsc

N = 10000
E = 90000
C = 14
HID = 128
NCLS = 25
NL = 3
ROUNDS = 3

XW = 48
RADW = 256
AXW = HID + 128

NWORK = 32
CH = 128
CPT = 22
EP = NWORK * CH * CPT
NT = 10240
STRIPE = NT // 16

ET = 512
NTC = 2000

_f32 = jnp.float32


def _mm(a, b):
    return lax.dot_general(a, b, (((a.ndim - 1,), (0,)), ((), ())),
                           preferred_element_type=_f32)


def _silu(x):
    return x * (1.0 / (1.0 + jnp.exp(-x)))


def _np_expand_consts():
    ru = np.zeros((XW, 3 * RADW), np.float32)
    rv = np.zeros((XW, 3 * RADW), np.float32)
    for i in range(3):
        for c in range(C):
            for d in range(C):
                ru[c * 3 + i, i * RADW + c * C + d] = 1.0
                rv[d * 3 + i, i * RADW + c * C + d] = 1.0
    re = np.zeros((16, XW), np.float32)
    for c in range(C):
        for i in range(3):
            re[c, c * 3 + i] = 1.0
    return ru, rv, re


_RU_NP, _RV_NP, _RE_NP = _np_expand_consts()


@functools.cache
def _sc_mesh():
    return plsc.VectorSubcoreMesh(core_axis_name="c", subcore_axis_name="s")


def _sc_gather(AX, BX, srcg, dstg):
    @functools.partial(
        pl.kernel,
        out_type=[
            jax.ShapeDtypeStruct((EP, AXW), _f32),
            jax.ShapeDtypeStruct((EP, AXW), _f32),
        ],
        mesh=_sc_mesh(),
        scratch_types=[
            pltpu.VMEM((CH,), jnp.int32),
            pltpu.VMEM((CH,), jnp.int32),
            pltpu.VMEM((CH, AXW), _f32),
            pltpu.VMEM((CH, AXW), _f32),
            pltpu.SemaphoreType.DMA,
        ],
        name="sc_gather",
    )
    def k(ax_h, bx_h, srcg_h, dstg_h, oa, ob, isrc, idst, ba, bb, sem):
        w = lax.axis_index("c") * 16 + lax.axis_index("s")

        @pl.loop(0, CPT)
        def chunk(j):
            pltpu.sync_copy(srcg_h.at[w, j], isrc)
            pltpu.sync_copy(dstg_h.at[w, j], idst)
            d1 = pltpu.async_copy(ax_h.at[isrc], ba, sem)
            d2 = pltpu.async_copy(bx_h.at[idst], bb, sem)
            d1.wait(); d2.wait()
            e0 = (w * CPT + j) * CH
            w1 = pltpu.async_copy(ba, oa.at[pl.ds(e0, CH)], sem)
            w2 = pltpu.async_copy(bb, ob.at[pl.ds(e0, CH)], sem)
            w1.wait(); w2.wait()

    return k(AX, BX, srcg, dstg)


def _sc_scatter(Mv, WX, dsts):
    @functools.partial(
        pl.kernel,
        out_type=[
            jax.ShapeDtypeStruct((2 * NT, HID), _f32),
            jax.ShapeDtypeStruct((2 * NT, HID), _f32),
        ],
        mesh=_sc_mesh(),
        scratch_types=[
            pltpu.VMEM_SHARED((NT, HID), _f32),
            pltpu.VMEM((CH,), jnp.int32),
            pltpu.VMEM((CH, HID), _f32),
            pltpu.VMEM((CH, HID), _f32),
            pltpu.SemaphoreType.DMA,
        ],
        name="sc_scatter",
    )
    def k(m_h, wx_h, dsts_h, oh, ox, acc, idx, buf, zbuf, sem):
        cid = lax.axis_index("c")
        sid = lax.axis_index("s")
        w = cid * 16 + sid
        r0 = sid * STRIPE
        z16 = jnp.zeros((16,), _f32)

        @pl.loop(0, CH * HID // 16)
        def zf(kk):
            zbuf[kk // (HID // 16), pl.ds((kk % (HID // 16)) * 16, 16)] = z16

        def zero_stripe():
            @pl.loop(0, STRIPE // CH)
            def zs(kk):
                pltpu.sync_copy(zbuf, acc.at[pl.ds(r0 + kk * CH, CH)])

        def scatter_phase(src_h):
            @pl.loop(0, CPT)
            def chunk(j):
                pltpu.sync_copy(dsts_h.at[w, j], idx)
                e0 = (w * CPT + j) * CH
                pltpu.async_copy(src_h.at[pl.ds(e0, CH)], buf, sem).wait()
                pltpu.sync_copy(buf, acc.at[idx], add=True)

        def readout(out_h):
            @pl.loop(0, STRIPE // CH)
            def ro(kk):
                pltpu.sync_copy(acc.at[pl.ds(r0 + kk * CH, CH)], buf)
                pltpu.sync_copy(buf, out_h.at[pl.ds(cid * NT + r0 + kk * CH,
                                                    CH)])

        zero_stripe()
        plsc.subcore_barrier()
        scatter_phase(m_h)
        plsc.subcore_barrier()
        readout(oh)
        plsc.subcore_barrier()
        zero_stripe()
        plsc.subcore_barrier()
        scatter_phase(wx_h)
        plsc.subcore_barrier()
        readout(ox)

    return k(Mv, WX, dsts)


def _full(shape):
    return pl.BlockSpec(shape, lambda i: (0,) * len(shape))


def _rows(bshape):
    return pl.BlockSpec(bshape, lambda i: (i,) + (0,) * (len(bshape) - 1))


def _axbx(a, b, x):
    xp = jnp.concatenate([x, jnp.zeros((x.shape[0], 128 - XW), _f32)], axis=1)
    return (jnp.concatenate([a, xp], axis=1),
            jnp.concatenate([b, xp], axis=1))


def _tc_init(S, Xp, emb_p, we1a0, we1b0):
    def body(s_ref, x_ref, emb_ref, wa_ref, wb_ref, h_ref, ax_ref, bx_ref):
        s = s_ref[...]
        oh = (s == lax.broadcasted_iota(jnp.int32, (1, 32), 1)).astype(_f32)
        h = _mm(oh, emb_ref[...])
        h_ref[...] = h
        ax, bx = _axbx(_mm(h, wa_ref[...]), _mm(h, wb_ref[...]), x_ref[...])
        ax_ref[...] = ax
        bx_ref[...] = bx

    return pl.pallas_call(
        body,
        grid=(N // NTC,),
        in_specs=[_rows((NTC, 1)), _rows((NTC, XW)), _full((32, HID)),
                  _full((HID, HID)), _full((HID, HID))],
        out_specs=[_rows((NTC, HID)), _rows((NTC, AXW)), _rows((NTC, AXW))],
        out_shape=[jax.ShapeDtypeStruct((N, HID), _f32),
                   jax.ShapeDtypeStruct((N, AXW), _f32),
                   jax.ShapeDtypeStruct((N, AXW), _f32)],
    )(S, Xp, emb_p, we1a0, we1b0)


def _tc_edge(GA, GB, ru, rv, wradp, we1r, we2, wxp, re):
    def body(ga_ref, gb_ref, ru_ref, rv_ref, wrad_ref,
             we1r_ref, we2_ref, wx_ref, re_ref, m_ref, wx_out_ref):
        ga = ga_ref[...]
        gb = gb_ref[...]
        xd = ga[:, HID:HID + XW] - gb[:, HID:HID + XW]
        p = _mm(xd, ru_ref[...])
        q = _mm(xd, rv_ref[...])
        rad = (p[:, :RADW] * q[:, :RADW]
               + p[:, RADW:2 * RADW] * q[:, RADW:2 * RADW]
               + p[:, 2 * RADW:] * q[:, 2 * RADW:])
        radn = rad / (1.0 + jnp.abs(rad))
        r = _silu(_mm(radn, wrad_ref[...]))
        m1 = _silu(ga[:, :HID] + gb[:, :HID] + _mm(r, we1r_ref[...]))
        m = _silu(_mm(m1, we2_ref[...]))
        cw = jnp.tanh(_mm(m, wx_ref[...]))
        m_ref[...] = m
        wxd = xd * _mm(cw, re_ref[...])
        wx_out_ref[...] = jnp.concatenate(
            [wxd, jnp.zeros((ET, 112 - XW), _f32), jnp.ones((ET, 16), _f32)],
            axis=1)

    return pl.pallas_call(
        body,
        grid=(EP // ET,),
        in_specs=[_rows((ET, AXW)), _rows((ET, AXW)),
                  _full((XW, 3 * RADW)), _full((XW, 3 * RADW)),
                  _full((RADW, HID)), _full((HID, HID)), _full((HID, HID)),
                  _full((HID, 16)), _full((16, XW))],
        out_specs=[_rows((ET, HID)), _rows((ET, HID))],
        out_shape=[jax.ShapeDtypeStruct((EP, HID), _f32),
                   jax.ShapeDtypeStruct((EP, HID), _f32)],
    )(GA, GB, ru, rv, wradp, we1r, we2, wxp, re)


def _node_common(h_ref, h0_ref, h1_ref, wh1a_ref, wh1b_ref, wh2_ref):
    h = h_ref[...]
    hagg = h0_ref[...] + h1_ref[...]
    t = _silu(_mm(h, wh1a_ref[...]) + _mm(hagg, wh1b_ref[...]))
    return h + _mm(t, wh2_ref[...])


def _x_common(x_ref, x0_ref, x1_ref):
    xq = x0_ref[...] + x1_ref[...]
    deg = xq[:, 112:113]
    return x_ref[...] + xq[:, :XW] / (deg + 1.0)


def _tc_node_mid(h, hp0, hp1, x, xp0, xp1, wh1a, wh1b, wh2, we1an, we1bn):
    def body(h_ref, h0_ref, h1_ref, x_ref, x0_ref, x1_ref,
             wh1a_ref, wh1b_ref, wh2_ref, wan_ref, wbn_ref,
             ho_ref, xo_ref, ax_ref, bx_ref):
        hn = _node_common(h_ref, h0_ref, h1_ref, wh1a_ref, wh1b_ref, wh2_ref)
        xn = _x_common(x_ref, x0_ref, x1_ref)
        ho_ref[...] = hn
        xo_ref[...] = xn
        ax, bx = _axbx(_mm(hn, wan_ref[...]), _mm(hn, wbn_ref[...]), xn)
        ax_ref[...] = ax
        bx_ref[...] = bx

    return pl.pallas_call(
        body,
        grid=(N // NTC,),
        in_specs=[_rows((NTC, HID))] * 3 + [_rows((NTC, XW))]
                 + [_rows((NTC, HID))] * 2 + [_full((HID, HID))] * 5,
        out_specs=[_rows((NTC, HID)), _rows((NTC, XW)),
                   _rows((NTC, AXW)), _rows((NTC, AXW))],
        out_shape=[jax.ShapeDtypeStruct((N, HID), _f32),
                   jax.ShapeDtypeStruct((N, XW), _f32),
                   jax.ShapeDtypeStruct((N, AXW), _f32),
                   jax.ShapeDtypeStruct((N, AXW), _f32)],
    )(h, hp0, hp1, x, xp0, xp1, wh1a, wh1b, wh2, we1an, we1bn)


def _tc_node_round(h, hp0, hp1, x, xp0, xp1, S, emb_p,
                   wh1a, wh1b, wh2, wm1, wm2, we1a0, we1b0):
    def body(h_ref, h0_ref, h1_ref, x_ref, x0_ref, x1_ref, s_ref, emb_ref,
             wh1a_ref, wh1b_ref, wh2_ref, wm1_ref, wm2_ref,
             wa_ref, wb_ref, ho_ref, xo_ref, ax_ref, bx_ref):
        hn = _node_common(h_ref, h0_ref, h1_ref, wh1a_ref, wh1b_ref, wh2_ref)
        xn = _x_common(x_ref, x0_ref, x1_ref)
        xo_ref[...] = xn
        mem = _mm(_silu(_mm(_silu(hn), wm1_ref[...])), wm2_ref[...])
        oh = (s_ref[...] == lax.broadcasted_iota(jnp.int32, (1, 32), 1)
              ).astype(_f32)
        hnew = _mm(oh, emb_ref[...]) + mem
        ho_ref[...] = hnew
        ax, bx = _axbx(_mm(hnew, wa_ref[...]), _mm(hnew, wb_ref[...]), xn)
        ax_ref[...] = ax
        bx_ref[...] = bx

    return pl.pallas_call(
        body,
        grid=(N // NTC,),
        in_specs=[_rows((NTC, HID))] * 3 + [_rows((NTC, XW))]
                 + [_rows((NTC, HID))] * 2 + [_rows((NTC, 1))]
                 + [_full((32, HID))] + [_full((HID, HID))] * 7,
        out_specs=[_rows((NTC, HID)), _rows((NTC, XW)),
                   _rows((NTC, AXW)), _rows((NTC, AXW))],
        out_shape=[jax.ShapeDtypeStruct((N, HID), _f32),
                   jax.ShapeDtypeStruct((N, XW), _f32),
                   jax.ShapeDtypeStruct((N, AXW), _f32),
                   jax.ShapeDtypeStruct((N, AXW), _f32)],
    )(h, hp0, hp1, x, xp0, xp1, S, emb_p,
      wh1a, wh1b, wh2, wm1, wm2, we1a0, we1b0)


def _tc_node_final(h, hp0, hp1, wh1a, wh1b, wh2, wr1, wr2):
    def body(h_ref, h0_ref, h1_ref, wh1a_ref, wh1b_ref, wh2_ref,
             wr1_ref, wr2_ref, o_ref):
        hn = _node_common(h_ref, h0_ref, h1_ref, wh1a_ref, wh1b_ref, wh2_ref)
        o_ref[...] = _mm(_silu(_mm(_silu(hn), wr1_ref[...])), wr2_ref[...])

    return pl.pallas_call(
        body,
        grid=(N // NTC,),
        in_specs=[_rows((NTC, HID))] * 3 + [_full((HID, HID))] * 4
                 + [_full((HID, NCLS))],
        out_specs=_rows((NTC, NCLS)),
        out_shape=jax.ShapeDtypeStruct((N, NCLS), _f32),
    )(h, hp0, hp1, wh1a, wh1b, wh2, wr1, wr2)


def kernel(X, S, edge_index, emb, W_rad, W_e1, W_e2, W_x, W_h1, W_h2,
           W_m1, W_m2, W_r1, W_r2):
    ru = jnp.asarray(_RU_NP)
    rv = jnp.asarray(_RV_NP)
    re = jnp.asarray(_RE_NP)

    Xp = jnp.pad(X.reshape(N, C * 3), ((0, 0), (0, XW - C * 3)))
    emb_p = jnp.pad(emb, ((0, 32 - NCLS), (0, 0)))
    S32 = S.astype(jnp.int32).reshape(N, 1)

    src = edge_index[0].astype(jnp.int32)
    dst = edge_index[1].astype(jnp.int32)
    padn = EP - E
    srcg = jnp.concatenate([src, jnp.zeros((padn,), jnp.int32)]
                           ).reshape(NWORK, CPT, CH)
    dstg = jnp.concatenate([dst, jnp.zeros((padn,), jnp.int32)]
                           ).reshape(NWORK, CPT, CH)
    dsts = jnp.concatenate([dst, jnp.full((padn,), N, jnp.int32)]
                           ).reshape(NWORK, CPT, CH)

    wradp = [jnp.pad(W_rad[l], ((0, RADW - C * C), (0, 0))) for l in range(NL)]
    we1a = [W_e1[l, :HID] for l in range(NL)]
    we1b = [W_e1[l, HID:2 * HID] for l in range(NL)]
    we1r = [W_e1[l, 2 * HID:] for l in range(NL)]
    wxp = [jnp.pad(W_x[l], ((0, 0), (0, 16 - C))) for l in range(NL)]
    wh1a = [W_h1[l, :HID] for l in range(NL)]
    wh1b = [W_h1[l, HID:] for l in range(NL)]

    h, AX, BX = _tc_init(S32, Xp, emb_p, we1a[0], we1b[0])
    x = Xp
    logits = None
    for r in range(ROUNDS):
        for l in range(NL):
            GA, GB = _sc_gather(AX, BX, srcg, dstg)
            Mv, WX = _tc_edge(GA, GB, ru, rv, wradp[l],
                              we1r[l], W_e2[l], wxp[l], re)
            Hp, Xq = _sc_scatter(Mv, WX, dsts)
            hp0, hp1 = Hp[:N], Hp[NT:NT + N]
            xp0, xp1 = Xq[:N], Xq[NT:NT + N]
            last = l == NL - 1
            if not last:
                h, x, AX, BX = _tc_node_mid(h, hp0, hp1, x, xp0, xp1,
                                            wh1a[l], wh1b[l], W_h2[l],
                                            we1a[l + 1], we1b[l + 1])
            elif r < ROUNDS - 1:
                h, x, AX, BX = _tc_node_round(h, hp0, hp1, x, xp0, xp1,
                                              S32, emb_p, wh1a[l], wh1b[l],
                                              W_h2[l], W_m1, W_m2,
                                              we1a[0], we1b[0])
            else:
                logits = _tc_node_final(h, hp0, hp1, wh1a[l], wh1b[l],
                                        W_h2[l], W_r1, W_r2)
    return logits
